# baseline (speedup 1.0000x reference)
import jax
import jax.numpy as jnp
from jax import lax
from jax.experimental import pallas as pl
from jax.experimental.pallas import tpu as pltpu
from jax.experimental.pallas import tpu_sc as plsc

N_NODES = 10000
MAX_DEG = 32
D_FEAT = 128
_INV = 1.0 / MAX_DEG

_NW = 32
_T = 8
_K = 12
_NB = 3
_C = _T * _K
_SC_N = _NW * _C

_TC_N = N_NODES - _SC_N
_BN = 16


def _reduce_tile(buf, obuf):
    @plsc.parallel_loop(0, _T)
    def _node(n):
        for c in range(D_FEAT // 16):
            sl = pl.ds(c * 16, 16)
            acc = buf[n, 0, sl]
            for k in range(1, MAX_DEG):
                acc = acc + buf[n, k, sl]
            obuf[n, sl] = acc * _INV


def _sc_body(mail, out, buf0, buf1, buf2, ob0, ob1, ob2,
             sem0, sem1, sem2, osem0, osem1, osem2):
    w = lax.axis_index("s") * 2 + lax.axis_index("c")
    base = w * _C
    bufs = (buf0, buf1, buf2)
    obs = (ob0, ob1, ob2)
    sems = (sem0, sem1, sem2)
    osems = (osem0, osem1, osem2)

    for b in range(_NB):
        pltpu.async_copy(mail.at[pl.ds(base + b * _T, _T)], bufs[b], sems[b])

    def group(i, carry):
        t0 = i * _NB
        for b in range(_NB):
            t = t0 + b
            node0 = base + t * _T
            pltpu.make_async_copy(mail.at[pl.ds(node0, _T)], bufs[b], sems[b]).wait()

            @pl.when(i >= 1)
            def _():
                pltpu.make_async_copy(obs[b], out.at[pl.ds(node0, _T)], osems[b]).wait()

            _reduce_tile(bufs[b], obs[b])
            pltpu.async_copy(obs[b], out.at[pl.ds(node0, _T)], osems[b])

            @pl.when(t + _NB < _K)
            def _():
                pltpu.async_copy(
                    mail.at[pl.ds(node0 + _NB * _T, _T)], bufs[b], sems[b])
        return carry

    lax.fori_loop(0, _K // _NB, group, 0)

    for b in range(_NB):
        pltpu.make_async_copy(obs[b], out.at[pl.ds(base, _T)], osems[b]).wait()


def _sc_mean(mail_sc):
    mesh = plsc.VectorSubcoreMesh(core_axis_name="c", subcore_axis_name="s")
    f = pl.kernel(
        _sc_body,
        out_type=jax.ShapeDtypeStruct((_SC_N, D_FEAT), jnp.float32),
        mesh=mesh,
        scratch_types=[
            pltpu.VMEM((_T, MAX_DEG, D_FEAT), jnp.float32),
            pltpu.VMEM((_T, MAX_DEG, D_FEAT), jnp.float32),
            pltpu.VMEM((_T, MAX_DEG, D_FEAT), jnp.float32),
            pltpu.VMEM((_T, D_FEAT), jnp.float32),
            pltpu.VMEM((_T, D_FEAT), jnp.float32),
            pltpu.VMEM((_T, D_FEAT), jnp.float32),
            pltpu.SemaphoreType.DMA,
            pltpu.SemaphoreType.DMA,
            pltpu.SemaphoreType.DMA,
            pltpu.SemaphoreType.DMA,
            pltpu.SemaphoreType.DMA,
            pltpu.SemaphoreType.DMA,
        ],
    )
    return f(mail_sc)


def _tc_body(x_ref, o_ref):
    o_ref[...] = jnp.sum(x_ref[...], axis=1) * _INV


def _tc_mean(mail_tc):
    return pl.pallas_call(
        _tc_body,
        grid=(_TC_N // _BN,),
        in_specs=[pl.BlockSpec((_BN, MAX_DEG, D_FEAT), lambda i: (i, 0, 0))],
        out_specs=pl.BlockSpec((_BN, D_FEAT), lambda i: (i, 0)),
        out_shape=jax.ShapeDtypeStruct((_TC_N, D_FEAT), jnp.float32),
    )(mail_tc)


def kernel(mailbox_m):
    sc_out = _sc_mean(mailbox_m[:_SC_N])
    tc_out = _tc_mean(mailbox_m[_SC_N:])
    return jnp.concatenate([sc_out, tc_out], axis=0)


# speedup vs baseline: 1.4418x; 1.4418x over previous
import jax
import jax.numpy as jnp
from jax import lax
from jax.experimental import pallas as pl
from jax.experimental.pallas import tpu as pltpu
from jax.experimental.pallas import tpu_sc as plsc

N_NODES = 10000
MAX_DEG = 32
D_FEAT = 128
_INV = 1.0 / MAX_DEG

_NW = 32
_T = 8
_K = 12
_NB = 3
_C = _T * _K
_SC_N = _NW * _C

_TC_N = N_NODES - _SC_N
_BN = 16


def _reduce_tile(buf, obuf):
    @plsc.parallel_loop(0, _T)
    def _node(n):
        for c in range(D_FEAT // 16):
            sl = pl.ds(c * 16, 16)
            acc = buf[n, 0, sl]
            for k in range(1, MAX_DEG):
                acc = acc + buf[n, k, sl]
            obuf[n, sl] = acc * _INV


def _sc_body(mail, out, buf0, buf1, buf2, ob0, ob1, ob2,
             sem0, sem1, sem2, osem0, osem1, osem2):
    w = lax.axis_index("s") * 2 + lax.axis_index("c")
    base = w * _C
    bufs = (buf0, buf1, buf2)
    obs = (ob0, ob1, ob2)
    sems = (sem0, sem1, sem2)
    osems = (osem0, osem1, osem2)

    for b in range(_NB):
        pltpu.async_copy(mail.at[pl.ds(base + b * _T, _T)], bufs[b], sems[b])

    def group(i, carry):
        t0 = i * _NB
        for b in range(_NB):
            t = t0 + b
            node0 = base + t * _T
            pltpu.make_async_copy(mail.at[pl.ds(node0, _T)], bufs[b], sems[b]).wait()

            @pl.when(i >= 1)
            def _():
                pltpu.make_async_copy(obs[b], out.at[pl.ds(node0, _T)], osems[b]).wait()

            _reduce_tile(bufs[b], obs[b])
            pltpu.async_copy(obs[b], out.at[pl.ds(node0, _T)], osems[b])

            @pl.when(t + _NB < _K)
            def _():
                pltpu.async_copy(
                    mail.at[pl.ds(node0 + _NB * _T, _T)], bufs[b], sems[b])
        return carry

    lax.fori_loop(0, _K // _NB, group, 0)

    for b in range(_NB):
        pltpu.make_async_copy(obs[b], out.at[pl.ds(base, _T)], osems[b]).wait()


def _sc_mean(mail):
    mesh = plsc.VectorSubcoreMesh(core_axis_name="c", subcore_axis_name="s")
    f = pl.kernel(
        _sc_body,
        out_type=jax.ShapeDtypeStruct((_SC_N, D_FEAT), jnp.float32),
        mesh=mesh,
        scratch_types=[
            pltpu.VMEM((_T, MAX_DEG, D_FEAT), jnp.float32),
            pltpu.VMEM((_T, MAX_DEG, D_FEAT), jnp.float32),
            pltpu.VMEM((_T, MAX_DEG, D_FEAT), jnp.float32),
            pltpu.VMEM((_T, D_FEAT), jnp.float32),
            pltpu.VMEM((_T, D_FEAT), jnp.float32),
            pltpu.VMEM((_T, D_FEAT), jnp.float32),
            pltpu.SemaphoreType.DMA,
            pltpu.SemaphoreType.DMA,
            pltpu.SemaphoreType.DMA,
            pltpu.SemaphoreType.DMA,
            pltpu.SemaphoreType.DMA,
            pltpu.SemaphoreType.DMA,
        ],
    )
    return f(mail)


def _tc_body(x_ref, o_ref):
    o_ref[...] = jnp.sum(x_ref[...], axis=1) * _INV


_BOFF = _SC_N // _BN


def _tc_mean(mail):
    return pl.pallas_call(
        _tc_body,
        grid=(_TC_N // _BN,),
        in_specs=[pl.BlockSpec((_BN, MAX_DEG, D_FEAT),
                               lambda i: (i + _BOFF, 0, 0))],
        out_specs=pl.BlockSpec((_BN, D_FEAT), lambda i: (i, 0)),
        out_shape=jax.ShapeDtypeStruct((_TC_N, D_FEAT), jnp.float32),
    )(mail)


def kernel(mailbox_m):
    sc_out = _sc_mean(mailbox_m)
    tc_out = _tc_mean(mailbox_m)
    return jnp.concatenate([sc_out, tc_out], axis=0)


# speedup vs baseline: 5.1268x; 3.5559x over previous
import jax
import jax.numpy as jnp
from jax import lax
from jax.experimental import pallas as pl
from jax.experimental.pallas import tpu as pltpu
from jax.experimental.pallas import tpu_sc as plsc

N_NODES = 10000
MAX_DEG = 32
D_FEAT = 128
_INV = 1.0 / MAX_DEG

_NW = 32
_T = 8
_K = 9
_NB = 3
_C = _T * _K
_SC_N = _NW * _C

_TC_N = N_NODES - _SC_N
_BN = 592


def _reduce_tile(buf, obuf):
    @plsc.parallel_loop(0, _T)
    def _node(n):
        for c in range(D_FEAT // 16):
            sl = pl.ds(c * 16, 16)
            acc = buf[n, 0, sl]
            for k in range(1, MAX_DEG):
                acc = acc + buf[n, k, sl]
            obuf[n, sl] = acc * _INV


def _sc_body(mail, out, buf0, buf1, buf2, ob0, ob1, ob2,
             sem0, sem1, sem2, osem0, osem1, osem2):
    w = lax.axis_index("s") * 2 + lax.axis_index("c")
    ibase = _TC_N + w * _C
    base = w * _C
    bufs = (buf0, buf1, buf2)
    obs = (ob0, ob1, ob2)
    sems = (sem0, sem1, sem2)
    osems = (osem0, osem1, osem2)

    for b in range(_NB):
        pltpu.async_copy(mail.at[pl.ds(ibase + b * _T, _T)], bufs[b], sems[b])

    def group(i, carry):
        t0 = i * _NB
        for b in range(_NB):
            t = t0 + b
            node0 = base + t * _T
            inode0 = ibase + t * _T
            pltpu.make_async_copy(mail.at[pl.ds(inode0, _T)], bufs[b], sems[b]).wait()

            @pl.when(i >= 1)
            def _():
                pltpu.make_async_copy(obs[b], out.at[pl.ds(node0, _T)], osems[b]).wait()

            _reduce_tile(bufs[b], obs[b])
            pltpu.async_copy(obs[b], out.at[pl.ds(node0, _T)], osems[b])

            @pl.when(t + _NB < _K)
            def _():
                pltpu.async_copy(
                    mail.at[pl.ds(inode0 + _NB * _T, _T)], bufs[b], sems[b])
        return carry

    lax.fori_loop(0, _K // _NB, group, 0)

    for b in range(_NB):
        pltpu.make_async_copy(obs[b], out.at[pl.ds(base, _T)], osems[b]).wait()


def _sc_mean(mail):
    mesh = plsc.VectorSubcoreMesh(core_axis_name="c", subcore_axis_name="s")
    f = pl.kernel(
        _sc_body,
        out_type=jax.ShapeDtypeStruct((_SC_N, D_FEAT), jnp.float32),
        mesh=mesh,
        scratch_types=[
            pltpu.VMEM((_T, MAX_DEG, D_FEAT), jnp.float32),
            pltpu.VMEM((_T, MAX_DEG, D_FEAT), jnp.float32),
            pltpu.VMEM((_T, MAX_DEG, D_FEAT), jnp.float32),
            pltpu.VMEM((_T, D_FEAT), jnp.float32),
            pltpu.VMEM((_T, D_FEAT), jnp.float32),
            pltpu.VMEM((_T, D_FEAT), jnp.float32),
            pltpu.SemaphoreType.DMA,
            pltpu.SemaphoreType.DMA,
            pltpu.SemaphoreType.DMA,
            pltpu.SemaphoreType.DMA,
            pltpu.SemaphoreType.DMA,
            pltpu.SemaphoreType.DMA,
        ],
    )
    return f(mail)


def _tc_body(x_ref, o_ref):
    o_ref[...] = jnp.sum(x_ref[...], axis=1) * _INV


def _tc_mean(mail):
    return pl.pallas_call(
        _tc_body,
        grid=(_TC_N // _BN,),
        in_specs=[pl.BlockSpec((_BN, MAX_DEG, D_FEAT),
                               lambda i: (i, 0, 0))],
        out_specs=pl.BlockSpec((_BN, D_FEAT), lambda i: (i, 0)),
        out_shape=jax.ShapeDtypeStruct((_TC_N, D_FEAT), jnp.float32),
    )(mail)


def kernel(mailbox_m):
    sc_out = _sc_mean(mailbox_m)
    tc_out = _tc_mean(mailbox_m)
    return jnp.concatenate([tc_out, sc_out], axis=0)


# speedup vs baseline: 5.1566x; 1.0058x over previous
import jax
import jax.numpy as jnp
from jax import lax
from jax.experimental import pallas as pl
from jax.experimental.pallas import tpu as pltpu
from jax.experimental.pallas import tpu_sc as plsc

N_NODES = 10000
MAX_DEG = 32
D_FEAT = 128
_INV = 1.0 / MAX_DEG

_NW = 32
_T = 8
_K = 6
_NB = 3
_C = _T * _K
_SC_N = _NW * _C

_TC_N = N_NODES - _SC_N
_BN = 368


def _reduce_tile(buf, obuf):
    @plsc.parallel_loop(0, _T)
    def _node(n):
        for c in range(D_FEAT // 16):
            sl = pl.ds(c * 16, 16)
            acc = buf[n, 0, sl]
            for k in range(1, MAX_DEG):
                acc = acc + buf[n, k, sl]
            obuf[n, sl] = acc * _INV


def _sc_body(mail, out, buf0, buf1, buf2, ob0, ob1, ob2,
             sem0, sem1, sem2, osem0, osem1, osem2):
    w = lax.axis_index("s") * 2 + lax.axis_index("c")
    ibase = _TC_N + w * _C
    base = w * _C
    bufs = (buf0, buf1, buf2)
    obs = (ob0, ob1, ob2)
    sems = (sem0, sem1, sem2)
    osems = (osem0, osem1, osem2)

    for b in range(_NB):
        pltpu.async_copy(mail.at[pl.ds(ibase + b * _T, _T)], bufs[b], sems[b])

    def group(i, carry):
        t0 = i * _NB
        for b in range(_NB):
            t = t0 + b
            node0 = base + t * _T
            inode0 = ibase + t * _T
            pltpu.make_async_copy(mail.at[pl.ds(inode0, _T)], bufs[b], sems[b]).wait()

            @pl.when(i >= 1)
            def _():
                pltpu.make_async_copy(obs[b], out.at[pl.ds(node0, _T)], osems[b]).wait()

            _reduce_tile(bufs[b], obs[b])
            pltpu.async_copy(obs[b], out.at[pl.ds(node0, _T)], osems[b])

            @pl.when(t + _NB < _K)
            def _():
                pltpu.async_copy(
                    mail.at[pl.ds(inode0 + _NB * _T, _T)], bufs[b], sems[b])
        return carry

    lax.fori_loop(0, _K // _NB, group, 0)

    for b in range(_NB):
        pltpu.make_async_copy(obs[b], out.at[pl.ds(base, _T)], osems[b]).wait()


def _sc_mean(mail):
    mesh = plsc.VectorSubcoreMesh(core_axis_name="c", subcore_axis_name="s")
    f = pl.kernel(
        _sc_body,
        out_type=jax.ShapeDtypeStruct((_SC_N, D_FEAT), jnp.float32),
        mesh=mesh,
        scratch_types=[
            pltpu.VMEM((_T, MAX_DEG, D_FEAT), jnp.float32),
            pltpu.VMEM((_T, MAX_DEG, D_FEAT), jnp.float32),
            pltpu.VMEM((_T, MAX_DEG, D_FEAT), jnp.float32),
            pltpu.VMEM((_T, D_FEAT), jnp.float32),
            pltpu.VMEM((_T, D_FEAT), jnp.float32),
            pltpu.VMEM((_T, D_FEAT), jnp.float32),
            pltpu.SemaphoreType.DMA,
            pltpu.SemaphoreType.DMA,
            pltpu.SemaphoreType.DMA,
            pltpu.SemaphoreType.DMA,
            pltpu.SemaphoreType.DMA,
            pltpu.SemaphoreType.DMA,
        ],
    )
    return f(mail)


def _tc_body(x_ref, o_ref):
    o_ref[...] = jnp.sum(x_ref[...], axis=1) * _INV


def _tc_mean(mail):
    return pl.pallas_call(
        _tc_body,
        grid=(_TC_N // _BN,),
        in_specs=[pl.BlockSpec((_BN, MAX_DEG, D_FEAT),
                               lambda i: (i, 0, 0))],
        out_specs=pl.BlockSpec((_BN, D_FEAT), lambda i: (i, 0)),
        out_shape=jax.ShapeDtypeStruct((_TC_N, D_FEAT), jnp.float32),
    )(mail)


def kernel(mailbox_m):
    sc_out = _sc_mean(mailbox_m)
    tc_out = _tc_mean(mailbox_m)
    return jnp.concatenate([tc_out, sc_out], axis=0)


# speedup vs baseline: 5.1605x; 1.0008x over previous
import jax
import jax.numpy as jnp
from jax import lax
from jax.experimental import pallas as pl
from jax.experimental.pallas import tpu as pltpu
from jax.experimental.pallas import tpu_sc as plsc

N_NODES = 10000
MAX_DEG = 32
D_FEAT = 128
_INV = 1.0 / MAX_DEG

_NW = 32
_T = 8
_K = 6
_NB = 3
_C = _T * _K
_SC_N = _NW * _C

_TC_N = N_NODES - _SC_N
_BN = 368


def _reduce_tile(buf, obuf):
    @plsc.parallel_loop(0, _T)
    def _node(n):
        for c in range(D_FEAT // 16):
            sl = pl.ds(c * 16, 16)
            acc = buf[n, 0, sl]
            for k in range(1, MAX_DEG):
                acc = acc + buf[n, k, sl]
            obuf[n, sl] = acc * _INV


def _sc_body(mail, out, buf0, buf1, buf2, ob0, ob1, ob2,
             sem0, sem1, sem2, osem0, osem1, osem2):
    w = lax.axis_index("s") * 2 + lax.axis_index("c")
    ibase = _TC_N + w * _C
    base = w * _C
    bufs = (buf0, buf1, buf2)
    obs = (ob0, ob1, ob2)
    sems = (sem0, sem1, sem2)
    osems = (osem0, osem1, osem2)

    for b in range(_NB):
        pltpu.async_copy(mail.at[pl.ds(ibase + b * _T, _T)], bufs[b], sems[b])

    def group(i, carry):
        t0 = i * _NB
        for b in range(_NB):
            t = t0 + b
            node0 = base + t * _T
            inode0 = ibase + t * _T
            pltpu.make_async_copy(mail.at[pl.ds(inode0, _T)], bufs[b], sems[b]).wait()

            @pl.when(i >= 1)
            def _():
                pltpu.make_async_copy(obs[b], out.at[pl.ds(node0, _T)], osems[b]).wait()

            _reduce_tile(bufs[b], obs[b])
            pltpu.async_copy(obs[b], out.at[pl.ds(node0, _T)], osems[b])

            @pl.when(t + _NB < _K)
            def _():
                pltpu.async_copy(
                    mail.at[pl.ds(inode0 + _NB * _T, _T)], bufs[b], sems[b])
        return carry

    lax.fori_loop(0, _K // _NB, group, 0)

    for b in range(_NB):
        pltpu.make_async_copy(obs[b], out.at[pl.ds(base, _T)], osems[b]).wait()


def _sc_mean(mail):
    mesh = plsc.VectorSubcoreMesh(core_axis_name="c", subcore_axis_name="s")
    f = pl.kernel(
        _sc_body,
        out_type=jax.ShapeDtypeStruct((_SC_N, D_FEAT), jnp.float32),
        mesh=mesh,
        scratch_types=[
            pltpu.VMEM((_T, MAX_DEG, D_FEAT), jnp.float32),
            pltpu.VMEM((_T, MAX_DEG, D_FEAT), jnp.float32),
            pltpu.VMEM((_T, MAX_DEG, D_FEAT), jnp.float32),
            pltpu.VMEM((_T, D_FEAT), jnp.float32),
            pltpu.VMEM((_T, D_FEAT), jnp.float32),
            pltpu.VMEM((_T, D_FEAT), jnp.float32),
            pltpu.SemaphoreType.DMA,
            pltpu.SemaphoreType.DMA,
            pltpu.SemaphoreType.DMA,
            pltpu.SemaphoreType.DMA,
            pltpu.SemaphoreType.DMA,
            pltpu.SemaphoreType.DMA,
        ],
    )
    return f(mail)


def _tc_body(x_ref, o_ref):
    o_ref[...] = jnp.sum(x_ref[...], axis=1) * _INV


def _tc_mean(mail):
    return pl.pallas_call(
        _tc_body,
        grid=(_TC_N // _BN,),
        in_specs=[pl.BlockSpec((_BN, MAX_DEG, D_FEAT),
                               lambda i: (i, 0, 0))],
        out_specs=pl.BlockSpec((_BN, D_FEAT), lambda i: (i, 0)),
        out_shape=jax.ShapeDtypeStruct((_TC_N, D_FEAT), jnp.float32),
    )(mail)


def kernel(mailbox_m):
    tc_out = _tc_mean(mailbox_m)
    sc_out = _sc_mean(mailbox_m)
    return jnp.concatenate([tc_out, sc_out], axis=0)


# speedup vs baseline: 7.0170x; 1.3597x over previous
import jax
import jax.numpy as jnp
from jax.experimental import pallas as pl

N_NODES = 10000
MAX_DEG = 32
D_FEAT = 128
_BN = 1000
_INV = 1.0 / MAX_DEG


def _mean_body(x_ref, o_ref):
    o_ref[...] = jnp.sum(x_ref[...], axis=1) * _INV


def kernel(mailbox_m):
    return pl.pallas_call(
        _mean_body,
        grid=(N_NODES // _BN,),
        in_specs=[pl.BlockSpec((_BN, MAX_DEG, D_FEAT), lambda i: (i, 0, 0))],
        out_specs=pl.BlockSpec((_BN, D_FEAT), lambda i: (i, 0)),
        out_shape=jax.ShapeDtypeStruct((N_NODES, D_FEAT), jnp.float32),
    )(mailbox_m)
